# baseline (speedup 1.0000x reference)
import functools

import jax
import jax.numpy as jnp
from jax import lax
from jax.experimental import pallas as pl
from jax.experimental.pallas import tpu as pltpu
from jax.experimental.pallas import tpu_sc as plsc


_BLOCK_ROWS = 32

_NC = 2
_NS = 16
_NW = _NC * _NS
_VIEW_COLS = 8192


def _season_kernel(x_ref, season_ref):
    i = pl.program_id(0)
    xb = x_ref[...]
    row = jax.lax.broadcasted_iota(jnp.int32, xb.shape, 0) + i * _BLOCK_ROWS
    season_ref[...] = jnp.where(row == 0, 0.0, xb)


def _trend_body(nrows_view, row0_chunks, x_hbm, out_hbm, zbuf, xbuf):
    wid = lax.axis_index("s") * _NC + lax.axis_index("c")

    def _fill(i, carry):
        zbuf[pl.ds(i * 16, 16)] = jnp.zeros((16,), jnp.float32)
        return carry

    lax.fori_loop(0, _VIEW_COLS // 16, _fill, 0)

    n_copies = nrows_view // _NW
    for j in range(n_copies):
        row = wid + _NW * j
        if j == 0:
            @pl.when(wid < row0_chunks)
            def _():
                pltpu.sync_copy(x_hbm.at[row], xbuf)
                pltpu.sync_copy(xbuf, out_hbm.at[row])

            @pl.when(wid >= row0_chunks)
            def _():
                pltpu.sync_copy(zbuf, out_hbm.at[row])
        else:
            pltpu.sync_copy(zbuf, out_hbm.at[row])


def _make_trend(nrows_view):
    row0_chunks = 32768 // _VIEW_COLS
    mesh = plsc.VectorSubcoreMesh(
        core_axis_name="c", subcore_axis_name="s",
        num_cores=_NC, num_subcores=_NS,
    )
    return pl.kernel(
        functools.partial(_trend_body, nrows_view, row0_chunks),
        out_type=jax.ShapeDtypeStruct((nrows_view, _VIEW_COLS), jnp.float32),
        mesh=mesh,
        scratch_types=[
            pltpu.VMEM((_VIEW_COLS,), jnp.float32),
            pltpu.VMEM((_VIEW_COLS,), jnp.float32),
        ],
    )


def kernel(x):
    rows, cols = x.shape
    nrows_view = rows * cols // _VIEW_COLS

    grid = (rows // _BLOCK_ROWS,)
    spec = pl.BlockSpec((_BLOCK_ROWS, cols), lambda i: (i, 0))
    season = pl.pallas_call(
        _season_kernel,
        grid=grid,
        in_specs=[spec],
        out_specs=spec,
        out_shape=jax.ShapeDtypeStruct((rows, cols), x.dtype),
        compiler_params=pltpu.CompilerParams(
            dimension_semantics=("parallel",),
        ),
    )(x)

    x_view = jnp.reshape(x, (nrows_view, _VIEW_COLS))
    trend = _make_trend(nrows_view)(x_view)
    trend = jnp.reshape(trend, (rows, cols))
    return (season, trend)


# speedup vs baseline: 3.3691x; 3.3691x over previous
import jax
import jax.numpy as jnp
from jax.experimental import pallas as pl
from jax.experimental.pallas import tpu as pltpu


_BLOCK_ROWS = 32


_BLOCK_COLS = 8192


def _split_kernel(x_ref, season_ref, trend_ref):
    i = pl.program_id(0)
    xb = x_ref[...]
    row = jax.lax.broadcasted_iota(jnp.int32, xb.shape, 0) + i * _BLOCK_ROWS
    is_row0 = row == 0
    season_ref[...] = jnp.where(is_row0, 0.0, xb)
    trend_ref[...] = jnp.where(is_row0, xb, 0.0)


def kernel(x):
    rows, cols = x.shape
    grid = (rows // _BLOCK_ROWS, cols // _BLOCK_COLS)
    spec = pl.BlockSpec((_BLOCK_ROWS, _BLOCK_COLS), lambda i, j: (i, j))
    season, trend = pl.pallas_call(
        _split_kernel,
        grid=grid,
        in_specs=[spec],
        out_specs=[spec, spec],
        out_shape=[
            jax.ShapeDtypeStruct((rows, cols), x.dtype),
            jax.ShapeDtypeStruct((rows, cols), x.dtype),
        ],
        compiler_params=pltpu.CompilerParams(
            dimension_semantics=("parallel", "parallel"),
        ),
    )(x)
    return (season, trend)


# speedup vs baseline: 4.3510x; 1.2915x over previous
import jax
import jax.numpy as jnp
from jax.experimental import pallas as pl
from jax.experimental.pallas import tpu as pltpu


_BLOCK_ROWS = 32


def _split_kernel(x_ref, season_ref, trend_ref):
    i = pl.program_id(0)
    xb = x_ref[...]
    row = jax.lax.broadcasted_iota(jnp.int32, xb.shape, 0) + i * _BLOCK_ROWS
    is_row0 = row == 0
    season_ref[...] = jnp.where(is_row0, 0.0, xb)
    trend_ref[...] = jnp.where(is_row0, xb, 0.0)


def kernel(x):
    rows, cols = x.shape
    grid = (rows // _BLOCK_ROWS,)
    spec = pl.BlockSpec((_BLOCK_ROWS, cols), lambda i: (i, 0))
    season, trend = pl.pallas_call(
        _split_kernel,
        grid=grid,
        in_specs=[spec],
        out_specs=[spec, spec],
        out_shape=[
            jax.ShapeDtypeStruct((rows, cols), x.dtype),
            jax.ShapeDtypeStruct((rows, cols), x.dtype),
        ],
        compiler_params=pltpu.CompilerParams(
            dimension_semantics=("parallel",),
        ),
    )(x)
    return (season, trend)
